# baseline (speedup 1.0000x reference)
import functools

import jax
import jax.numpy as jnp
from jax import lax
from jax.experimental import pallas as pl
from jax.experimental.pallas import tpu as pltpu
from jax.experimental.pallas import tpu_sc as plsc

NUM_ENTITIES = 100000
NUM_RELATIONS = 1000
DIM = 200
BATCH = 16384

NC, NS, L = 2, 16, 16
NW = NC * NS
B_PER_W = BATCH // NW
CHUNK = 32
NCHUNK = B_PER_W // CHUNK
NFULL = DIM // L
TAIL_OFF = DIM - L
TAIL_KEEP = L - (DIM - NFULL * L)
ACC_STRIDE = L + 1


def _score_chunk(rh, rt, rr, accbuf, out_v, out_base, tail_mask, lane):
    def body(i, carry):
        acc = jnp.zeros((L,), jnp.float32)
        for j in range(NFULL + 1):
            off = j * L if j < NFULL else TAIL_OFF
            h_re = rh[i, pl.ds(off, L)]
            h_im = rh[i, pl.ds(off + DIM, L)]
            t_re = rt[i, pl.ds(off, L)]
            t_im = rt[i, pl.ds(off + DIM, L)]
            r_re = rr[i, pl.ds(off, L)]
            r_im = rr[i, pl.ds(off + DIM, L)]
            p = h_re * t_re + h_im * t_im
            q = h_re * t_im - h_im * t_re
            term = r_re * p + r_im * q
            if j == NFULL:
                term = jnp.where(tail_mask, term, 0.0)
            acc = acc + term
        accbuf[pl.ds(i * ACC_STRIDE, L)] = acc
        return carry

    lax.fori_loop(0, CHUNK, body, 0)
    for k in range(0, CHUNK, L):
        base_idx = (lane + k) * ACC_STRIDE
        tot = jnp.zeros((L,), jnp.float32)
        for j in range(L):
            tot = tot + plsc.load_gather(accbuf, [base_idx + j])
        out_v[pl.ds(out_base + k, L)] = tot


def _complex_score_kernel(heads_hbm, rels_hbm, tails_hbm, ent_hbm, rel_hbm,
                          out_hbm, idx_h, idx_t, idx_r, rows_h, rows_t,
                          rows_r, accbuf, out_v, sem):
    wid = lax.axis_index("s") * NC + lax.axis_index("c")
    base = wid * B_PER_W
    lane = lax.iota(jnp.int32, L)
    tail_mask = lane >= TAIL_KEEP

    def chunk_body(g, carry):
        cbase = base + g * CHUNK
        pltpu.sync_copy(heads_hbm.at[pl.ds(cbase, CHUNK)], idx_h)
        pltpu.sync_copy(tails_hbm.at[pl.ds(cbase, CHUNK)], idx_t)
        pltpu.sync_copy(rels_hbm.at[pl.ds(cbase, CHUNK)], idx_r)
        c1 = pltpu.async_copy(ent_hbm.at[idx_h], rows_h, sem)
        c2 = pltpu.async_copy(ent_hbm.at[idx_t], rows_t, sem)
        c3 = pltpu.async_copy(rel_hbm.at[idx_r], rows_r, sem)
        c1.wait()
        c2.wait()
        c3.wait()
        _score_chunk(rows_h, rows_t, rows_r, accbuf, out_v, g * CHUNK,
                     tail_mask, lane)
        return carry

    lax.fori_loop(0, NCHUNK, chunk_body, 0)
    pltpu.sync_copy(out_v, out_hbm.at[pl.ds(base, B_PER_W)])


@jax.jit
def _compl_ex(heads, rels, tails, entity_emb, rel_emb):
    mesh = plsc.VectorSubcoreMesh(
        core_axis_name="c", subcore_axis_name="s", num_cores=NC,
        num_subcores=NS)
    run = functools.partial(
        pl.kernel,
        out_type=jax.ShapeDtypeStruct((BATCH,), jnp.float32),
        mesh=mesh,
        compiler_params=pltpu.CompilerParams(
            needs_layout_passes=False, use_tc_tiling_on_sc=False),
        scratch_types=[
            pltpu.VMEM((CHUNK,), jnp.int32),
            pltpu.VMEM((CHUNK,), jnp.int32),
            pltpu.VMEM((CHUNK,), jnp.int32),
            pltpu.VMEM((CHUNK, 2 * DIM), jnp.float32),
            pltpu.VMEM((CHUNK, 2 * DIM), jnp.float32),
            pltpu.VMEM((CHUNK, 2 * DIM), jnp.float32),
            pltpu.VMEM((CHUNK * ACC_STRIDE,), jnp.float32),
            pltpu.VMEM((B_PER_W,), jnp.float32),
            pltpu.SemaphoreType.DMA,
        ],
    )(_complex_score_kernel)
    return run(heads, rels, tails, entity_emb, rel_emb)


def kernel(heads, rels, tails, entity_emb, rel_emb):
    return _compl_ex(
        heads.astype(jnp.int32),
        rels.astype(jnp.int32),
        tails.astype(jnp.int32),
        entity_emb.astype(jnp.float32),
        rel_emb.astype(jnp.float32),
    )
